# Initial kernel scaffold; baseline (speedup 1.0000x reference)
import functools

import jax
import jax.numpy as jnp
from jax import lax
from jax.experimental import pallas as pl
from jax.experimental.pallas import tpu as pltpu
from jax.experimental.pallas import tpu_sc as plsc

B = 16384
NEG = 20
D = 64
NC = 2
NS = 16
NW = NC * NS
EPW = B // NW
CH = 64
NCHUNK = EPW // CH
NIDX = CH * NEG // 128


def _sc_scores(pos_u, pos_v, neg_v_flat, u_emb, v_emb):
    mesh = plsc.VectorSubcoreMesh(core_axis_name="c", subcore_axis_name="s")

    @functools.partial(
        pl.kernel,
        out_type=(
            jax.ShapeDtypeStruct((B,), jnp.float32),
            jax.ShapeDtypeStruct((NW * NCHUNK, NEG, CH), jnp.float32),
        ),
        mesh=mesh,
        scratch_types=[
            pltpu.VMEM((CH,), jnp.int32),
            pltpu.VMEM((CH,), jnp.int32),
            pltpu.VMEM((NIDX, 128), jnp.int32),
            pltpu.VMEM((CH, D), jnp.float32),
            pltpu.VMEM((CH, D), jnp.float32),
            pltpu.VMEM((CH * NEG, D), jnp.float32),
            pltpu.VMEM((CH,), jnp.float32),
            pltpu.VMEM((NEG, CH), jnp.float32),
            pltpu.SemaphoreType.DMA,
        ],
    )
    def kern(pos_u_h, pos_v_h, neg_h, u_h, v_h, pos_o, neg_o,
             idx_u, idx_v, idx_n, rows_u, rows_v, rows_n, out_p, out_n, sem):
        wid = lax.axis_index("s") * NC + lax.axis_index("c")
        wbase = pl.multiple_of(wid * EPW, EPW)
        for c in range(NCHUNK):
            base = wbase + c * CH
            pltpu.sync_copy(pos_u_h.at[pl.ds(base, CH)], idx_u)
            pltpu.sync_copy(pos_v_h.at[pl.ds(base, CH)], idx_v)
            nb = base * NEG
            for j in range(NIDX):
                pltpu.sync_copy(neg_h.at[pl.ds(nb + j * 128, 128)], idx_n.at[j])
            cps = [pltpu.async_copy(u_h.at[idx_u], rows_u, sem),
                   pltpu.async_copy(v_h.at[idx_v], rows_v, sem)]
            for j in range(NIDX):
                cps.append(pltpu.async_copy(
                    v_h.at[idx_n.at[j]], rows_n.at[pl.ds(j * 128, 128)], sem))
            for cp in cps:
                cp.wait()
            for g in range(CH // 16):
                row16 = g * 16 + lax.iota(jnp.int32, 16)
                row20 = row16 * NEG

                def body(d, carry, row16=row16, row20=row20):
                    accp, accn = carry
                    dcol = jnp.zeros((16,), jnp.int32) + d
                    gu = plsc.load_gather(rows_u, [row16, dcol])
                    gv = plsc.load_gather(rows_v, [row16, dcol])
                    accp = accp + gu * gv
                    accn = tuple(
                        a + gu * plsc.load_gather(rows_n, [row20 + n, dcol])
                        for n, a in enumerate(accn))
                    return accp, accn

                accp, accn = lax.fori_loop(
                    0, D, body,
                    (jnp.zeros((16,), jnp.float32),
                     tuple(jnp.zeros((16,), jnp.float32) for _ in range(NEG))))
                out_p[pl.ds(g * 16, 16)] = accp
                for n in range(NEG):
                    out_n[n, pl.ds(g * 16, 16)] = accn[n]
            pltpu.sync_copy(out_p, pos_o.at[pl.ds(base, CH)])
            pltpu.sync_copy(out_n, neg_o.at[wid * NCHUNK + c])

    return kern(pos_u, pos_v, neg_v_flat, u_emb, v_emb)


def _loss_body(pos_ref, neg_ref, out_ref):
    p = jnp.clip(pos_ref[...], -10.0, 10.0)
    lp = jnp.sum(jnp.log1p(jnp.exp(-p)))
    q = jnp.clip(neg_ref[...], -10.0, 10.0)
    ln = jnp.sum(jnp.log1p(jnp.exp(q)))
    out_ref[0, 0] = (lp + ln) * (1.0 / B)


def kernel(pos_u, pos_v, neg_v, u_emb, v_emb):
    pos_s, neg_s = _sc_scores(pos_u, pos_v, neg_v.reshape(-1), u_emb, v_emb)
    out = pl.pallas_call(
        _loss_body,
        out_shape=jax.ShapeDtypeStruct((1, 1), jnp.float32),
        out_specs=pl.BlockSpec(memory_space=pltpu.SMEM),
    )(pos_s.reshape(128, 128), neg_s.reshape(NW * NCHUNK * NEG, CH))
    return out[0, 0]


# speedup vs baseline: 3.9081x; 3.9081x over previous
import functools

import jax
import jax.numpy as jnp
from jax import lax
from jax.experimental import pallas as pl
from jax.experimental.pallas import tpu as pltpu
from jax.experimental.pallas import tpu_sc as plsc

B = 16384
NEG = 20
D = 64
NC = 2
NS = 16
NW = NC * NS
EPW = B // NW
CH = 64
NCHUNK = EPW // CH
NIDX = CH * NEG // 128


def _sc_scores(pos_u, pos_v, neg_v_flat, u_emb, v_emb):
    mesh = plsc.VectorSubcoreMesh(core_axis_name="c", subcore_axis_name="s")

    @functools.partial(
        pl.kernel,
        out_type=(
            jax.ShapeDtypeStruct((B,), jnp.float32),
            jax.ShapeDtypeStruct((NW * NCHUNK, NEG, CH), jnp.float32),
        ),
        mesh=mesh,
        compiler_params=pltpu.CompilerParams(needs_layout_passes=False, use_tc_tiling_on_sc=False),
        scratch_types=[
            pltpu.VMEM((CH,), jnp.int32),
            pltpu.VMEM((CH,), jnp.int32),
            pltpu.VMEM((NIDX, 128), jnp.int32),
            pltpu.VMEM((CH, D), jnp.float32),
            pltpu.VMEM((CH, D), jnp.float32),
            pltpu.VMEM((CH * NEG, D), jnp.float32),
            pltpu.VMEM((CH,), jnp.float32),
            pltpu.VMEM((NEG, CH), jnp.float32),
            pltpu.SemaphoreType.DMA,
        ],
    )
    def kern(pos_u_h, pos_v_h, neg_h, u_h, v_h, pos_o, neg_o,
             idx_u, idx_v, idx_n, rows_u, rows_v, rows_n, out_p, out_n, sem):
        wid = lax.axis_index("s") * NC + lax.axis_index("c")
        wbase = pl.multiple_of(wid * EPW, EPW)
        for c in range(NCHUNK):
            base = wbase + c * CH
            pltpu.sync_copy(pos_u_h.at[pl.ds(base, CH)], idx_u)
            pltpu.sync_copy(pos_v_h.at[pl.ds(base, CH)], idx_v)
            nb = base * NEG
            for j in range(NIDX):
                pltpu.sync_copy(neg_h.at[pl.ds(nb + j * 128, 128)], idx_n.at[j])
            cps = [pltpu.async_copy(u_h.at[idx_u], rows_u, sem),
                   pltpu.async_copy(v_h.at[idx_v], rows_v, sem)]
            for j in range(NIDX):
                cps.append(pltpu.async_copy(
                    v_h.at[idx_n.at[j]], rows_n.at[pl.ds(j * 128, 128)], sem))
            for cp in cps:
                cp.wait()
            for g in range(CH // 16):
                row16 = g * 16 + lax.iota(jnp.int32, 16)
                row20 = row16 * NEG

                def body(d, carry, row16=row16, row20=row20):
                    accp, accn = carry
                    dcol = jnp.zeros((16,), jnp.int32) + d
                    gu = plsc.load_gather(rows_u, [row16, dcol])
                    gv = plsc.load_gather(rows_v, [row16, dcol])
                    accp = accp + gu * gv
                    accn = tuple(
                        a + gu * plsc.load_gather(rows_n, [row20 + n, dcol])
                        for n, a in enumerate(accn))
                    return accp, accn

                accp, accn = lax.fori_loop(
                    0, D, body,
                    (jnp.zeros((16,), jnp.float32),
                     tuple(jnp.zeros((16,), jnp.float32) for _ in range(NEG))))
                out_p[pl.ds(g * 16, 16)] = accp
                for n in range(NEG):
                    out_n[n, pl.ds(g * 16, 16)] = accn[n]
            pltpu.sync_copy(out_p, pos_o.at[pl.ds(base, CH)])
            pltpu.sync_copy(out_n, neg_o.at[wid * NCHUNK + c])

    return kern(pos_u, pos_v, neg_v_flat, u_emb, v_emb)


def _loss_body(pos_ref, neg_ref, out_ref):
    p = jnp.clip(pos_ref[...], -10.0, 10.0)
    lp = jnp.sum(jnp.log1p(jnp.exp(-p)))
    q = jnp.clip(neg_ref[...], -10.0, 10.0)
    ln = jnp.sum(jnp.log1p(jnp.exp(q)))
    out_ref[0, 0] = (lp + ln) * (1.0 / B)


def kernel(pos_u, pos_v, neg_v, u_emb, v_emb):
    pos_s, neg_s = _sc_scores(pos_u, pos_v, neg_v.reshape(-1), u_emb, v_emb)
    out = pl.pallas_call(
        _loss_body,
        out_shape=jax.ShapeDtypeStruct((1, 1), jnp.float32),
        out_specs=pl.BlockSpec(memory_space=pltpu.SMEM),
    )(pos_s.reshape(128, 128), neg_s.reshape(NW * NCHUNK * NEG, CH))
    return out[0, 0]


# speedup vs baseline: 3.9685x; 1.0154x over previous
import functools

import jax
import jax.numpy as jnp
from jax import lax
from jax.experimental import pallas as pl
from jax.experimental.pallas import tpu as pltpu
from jax.experimental.pallas import tpu_sc as plsc

B = 16384
NEG = 20
D = 64
NC = 2
NS = 16
NW = NC * NS
EPW = B // NW
CH = 32
NCHUNK = EPW // CH
NIDX = CH * NEG // 128
NGRP = CH // 16


def _sc_scores(pos_u, pos_v, neg_v_flat, u_emb, v_emb):
    mesh = plsc.VectorSubcoreMesh(core_axis_name="c", subcore_axis_name="s")

    @functools.partial(
        pl.kernel,
        out_type=(
            jax.ShapeDtypeStruct((B,), jnp.float32),
            jax.ShapeDtypeStruct((NW * NCHUNK, NEG, CH), jnp.float32),
        ),
        mesh=mesh,
        compiler_params=pltpu.CompilerParams(
            needs_layout_passes=False, use_tc_tiling_on_sc=False),
        scratch_types=[
            pltpu.VMEM((EPW,), jnp.int32),
            pltpu.VMEM((EPW,), jnp.int32),
            pltpu.VMEM((EPW * NEG,), jnp.int32),
            pltpu.VMEM((2, CH, D), jnp.float32),
            pltpu.VMEM((2, CH, D), jnp.float32),
            pltpu.VMEM((2, CH * NEG, D), jnp.float32),
            pltpu.VMEM((CH,), jnp.float32),
            pltpu.VMEM((NEG, CH), jnp.float32),
            pltpu.SemaphoreType.DMA,
            pltpu.SemaphoreType.DMA,
        ],
    )
    def kern(pos_u_h, pos_v_h, neg_h, u_h, v_h, pos_o, neg_o,
             idx_u, idx_v, idx_n, rows_u, rows_v, rows_n, out_p, out_n,
             sem0, sem1):
        wid = lax.axis_index("s") * NC + lax.axis_index("c")
        wbase = pl.multiple_of(wid * EPW, EPW)
        pltpu.sync_copy(pos_u_h.at[pl.ds(wbase, EPW)], idx_u)
        pltpu.sync_copy(pos_v_h.at[pl.ds(wbase, EPW)], idx_v)
        pltpu.sync_copy(neg_h.at[pl.ds(wbase * NEG, EPW * NEG)], idx_n)
        sems = (sem0, sem1)

        def fire(c, b):
            off = pl.multiple_of(c * CH, CH)
            sem = sems[b]
            pltpu.async_copy(u_h.at[idx_u.at[pl.ds(off, CH)]],
                             rows_u.at[b], sem)
            pltpu.async_copy(v_h.at[idx_v.at[pl.ds(off, CH)]],
                             rows_v.at[b], sem)
            noff = pl.multiple_of(c * CH * NEG, CH * NEG)
            for j in range(NIDX):
                pltpu.async_copy(
                    v_h.at[idx_n.at[pl.ds(noff + j * 128, 128)]],
                    rows_n.at[b].at[pl.ds(j * 128, 128)], sem)

        def drain(b):
            sem = sems[b]
            pltpu.make_async_copy(u_h.at[pl.ds(0, CH)], rows_u.at[b], sem).wait()
            pltpu.make_async_copy(u_h.at[pl.ds(0, CH)], rows_v.at[b], sem).wait()
            pltpu.make_async_copy(u_h.at[pl.ds(0, CH * NEG)], rows_n.at[b],
                                  sem).wait()

        def compute(c, b):
            ru, rv, rn = rows_u.at[b], rows_v.at[b], rows_n.at[b]
            DT = 8
            for g in range(NGRP):
                row16 = g * 16 + lax.iota(jnp.int32, 16)
                row20 = row16 * NEG
                ones = jnp.ones((16,), jnp.int32)

                def dbody(t, carry, row16=row16, row20=row20, ones=ones,
                          ru=ru, rv=rv, rn=rn):
                    accp, accn, dvec = carry
                    accn = list(accn)
                    for _ in range(DT):
                        gu = plsc.load_gather(ru, [row16, dvec])
                        gv = plsc.load_gather(rv, [row16, dvec])
                        accp = accp + gu * gv
                        for n in range(NEG):
                            gn = plsc.load_gather(rn, [row20 + n, dvec])
                            accn[n] = accn[n] + gu * gn
                        dvec = dvec + ones
                    return accp, tuple(accn), dvec

                accp, accn, _ = lax.fori_loop(
                    0, D // DT, dbody,
                    (jnp.zeros((16,), jnp.float32),
                     tuple(jnp.zeros((16,), jnp.float32) for _ in range(NEG)),
                     jnp.zeros((16,), jnp.int32)))
                out_p[pl.ds(g * 16, 16)] = accp
                for n in range(NEG):
                    out_n[n, pl.ds(g * 16, 16)] = accn[n]
            pltpu.sync_copy(out_p, pos_o.at[pl.ds(wbase + c * CH, CH)])
            pltpu.sync_copy(out_n, neg_o.at[wid * NCHUNK + c])

        fire(0, 0)

        @pl.loop(0, NCHUNK, step=2)
        def _(c):
            drain(0)
            fire(c + 1, 1)
            compute(c, 0)
            drain(1)

            @pl.when(c + 2 < NCHUNK)
            def _():
                fire(c + 2, 0)

            compute(c + 1, 1)

    return kern(pos_u, pos_v, neg_v_flat, u_emb, v_emb)


def _loss_body(pos_ref, neg_ref, out_ref):
    p = jnp.clip(pos_ref[...], -10.0, 10.0)
    lp = jnp.sum(jnp.log1p(jnp.exp(-p)))
    q = jnp.clip(neg_ref[...], -10.0, 10.0)
    ln = jnp.sum(jnp.log1p(jnp.exp(q)))
    out_ref[0, 0] = (lp + ln) * (1.0 / B)


def kernel(pos_u, pos_v, neg_v, u_emb, v_emb):
    pos_s, neg_s = _sc_scores(pos_u, pos_v, neg_v.reshape(-1), u_emb, v_emb)
    out = pl.pallas_call(
        _loss_body,
        out_shape=jax.ShapeDtypeStruct((1, 1), jnp.float32),
        out_specs=pl.BlockSpec(memory_space=pltpu.SMEM),
    )(pos_s.reshape(128, 128), neg_s.reshape(NW * NCHUNK * NEG, CH))
    return out[0, 0]


# speedup vs baseline: 5.4406x; 1.3709x over previous
import functools

import jax
import jax.numpy as jnp
from jax import lax
from jax.experimental import pallas as pl
from jax.experimental.pallas import tpu as pltpu
from jax.experimental.pallas import tpu_sc as plsc

B = 16384
NEG = 20
D = 64
NC = 2
NS = 16
NW = NC * NS
EPW = B // NW
CH = 32
NCHUNK = EPW // CH
NIDX = CH * NEG // 128
NGRP = CH // 16


def _sc_scores(pos_u, pos_v, neg_v_flat, u_emb, v_emb):
    mesh = plsc.VectorSubcoreMesh(core_axis_name="c", subcore_axis_name="s")

    @functools.partial(
        pl.kernel,
        out_type=(
            jax.ShapeDtypeStruct((B,), jnp.float32),
            jax.ShapeDtypeStruct((NW * NCHUNK, NEG, CH), jnp.float32),
        ),
        mesh=mesh,
        compiler_params=pltpu.CompilerParams(
            needs_layout_passes=False, use_tc_tiling_on_sc=False),
        scratch_types=[
            pltpu.VMEM((EPW,), jnp.int32),
            pltpu.VMEM((EPW,), jnp.int32),
            pltpu.VMEM((EPW * NEG,), jnp.int32),
            pltpu.VMEM((2, CH, D), jnp.float32),
            pltpu.VMEM((2, CH, D), jnp.float32),
            pltpu.VMEM((2, CH * NEG, D), jnp.float32),
            pltpu.VMEM((CH,), jnp.float32),
            pltpu.VMEM((NEG, CH), jnp.float32),
            pltpu.SemaphoreType.DMA,
            pltpu.SemaphoreType.DMA,
        ],
    )
    def kern(pos_u_h, pos_v_h, neg_h, u_h, v_h, pos_o, neg_o,
             idx_u, idx_v, idx_n, rows_u, rows_v, rows_n, out_p, out_n,
             sem0, sem1):
        wid = lax.axis_index("s") * NC + lax.axis_index("c")
        wbase = pl.multiple_of(wid * EPW, EPW)
        pltpu.sync_copy(pos_u_h.at[pl.ds(wbase, EPW)], idx_u)
        pltpu.sync_copy(pos_v_h.at[pl.ds(wbase, EPW)], idx_v)
        pltpu.sync_copy(neg_h.at[pl.ds(wbase * NEG, EPW * NEG)], idx_n)
        sems = (sem0, sem1)

        def fire(c, b):
            off = pl.multiple_of(c * CH, CH)
            sem = sems[b]
            pltpu.async_copy(u_h.at[idx_u.at[pl.ds(off, CH)]],
                             rows_u.at[b], sem)
            pltpu.async_copy(v_h.at[idx_v.at[pl.ds(off, CH)]],
                             rows_v.at[b], sem)
            noff = pl.multiple_of(c * CH * NEG, CH * NEG)
            for j in range(NIDX):
                pltpu.async_copy(
                    v_h.at[idx_n.at[pl.ds(noff + j * 128, 128)]],
                    rows_n.at[b].at[pl.ds(j * 128, 128)], sem)

        def drain(b):
            sem = sems[b]
            pltpu.make_async_copy(u_h.at[pl.ds(0, CH)], rows_u.at[b], sem).wait()
            pltpu.make_async_copy(u_h.at[pl.ds(0, CH)], rows_v.at[b], sem).wait()
            pltpu.make_async_copy(u_h.at[pl.ds(0, CH * NEG)], rows_n.at[b],
                                  sem).wait()

        def compute(c, b):
            ru, rv, rn = rows_u.at[b], rows_v.at[b], rows_n.at[b]
            ilane = lax.iota(jnp.int32, 16)
            NK = D // 16
            for g in range(NGRP):

                def ebody(e, carry, ru=ru, rv=rv, rn=rn, g=g):
                    accp, accn = carry
                    accn = list(accn)
                    eg = g * 16 + e
                    sel = ilane == e
                    us = [ru[eg, pl.ds(16 * k, 16)] for k in range(NK)]
                    vs = [rv[eg, pl.ds(16 * k, 16)] for k in range(NK)]
                    pp = us[0] * vs[0]
                    for k in range(1, NK):
                        pp = pp + us[k] * vs[k]
                    accp = jnp.where(sel, jnp.sum(pp), accp)
                    for n in range(NEG):
                        ns = [rn[eg * NEG + n, pl.ds(16 * k, 16)]
                              for k in range(NK)]
                        qq = us[0] * ns[0]
                        for k in range(1, NK):
                            qq = qq + us[k] * ns[k]
                        accn[n] = jnp.where(sel, jnp.sum(qq), accn[n])
                    return accp, tuple(accn)

                accp, accn = lax.fori_loop(
                    0, 16, ebody,
                    (jnp.zeros((16,), jnp.float32),
                     tuple(jnp.zeros((16,), jnp.float32) for _ in range(NEG))))
                out_p[pl.ds(g * 16, 16)] = accp
                for n in range(NEG):
                    out_n[n, pl.ds(g * 16, 16)] = accn[n]
            pltpu.sync_copy(out_p, pos_o.at[pl.ds(wbase + c * CH, CH)])
            pltpu.sync_copy(out_n, neg_o.at[wid * NCHUNK + c])

        fire(0, 0)

        @pl.loop(0, NCHUNK, step=2)
        def _(c):
            drain(0)
            fire(c + 1, 1)
            compute(c, 0)
            drain(1)

            @pl.when(c + 2 < NCHUNK)
            def _():
                fire(c + 2, 0)

            compute(c + 1, 1)

    return kern(pos_u, pos_v, neg_v_flat, u_emb, v_emb)


def _loss_body(pos_ref, neg_ref, out_ref):
    p = jnp.clip(pos_ref[...], -10.0, 10.0)
    lp = jnp.sum(jnp.log1p(jnp.exp(-p)))
    q = jnp.clip(neg_ref[...], -10.0, 10.0)
    ln = jnp.sum(jnp.log1p(jnp.exp(q)))
    out_ref[0, 0] = (lp + ln) * (1.0 / B)


def kernel(pos_u, pos_v, neg_v, u_emb, v_emb):
    pos_s, neg_s = _sc_scores(pos_u, pos_v, neg_v.reshape(-1), u_emb, v_emb)
    out = pl.pallas_call(
        _loss_body,
        out_shape=jax.ShapeDtypeStruct((1, 1), jnp.float32),
        out_specs=pl.BlockSpec(memory_space=pltpu.SMEM),
    )(pos_s.reshape(128, 128), neg_s.reshape(NW * NCHUNK * NEG, CH))
    return out[0, 0]


# speedup vs baseline: 7.7233x; 1.4196x over previous
import functools

import jax
import jax.numpy as jnp
from jax import lax
from jax.experimental import pallas as pl
from jax.experimental.pallas import tpu as pltpu
from jax.experimental.pallas import tpu_sc as plsc

B = 16384
NEG = 20
D = 64
NC = 2
NS = 16
NW = NC * NS
EPW = B // NW
CH = 16
NCHUNK = EPW // CH
NGRP = CH // 16


def _sc_scores(pos_u, pos_v, neg_v_flat, u_emb, v_emb):
    mesh = plsc.VectorSubcoreMesh(core_axis_name="c", subcore_axis_name="s")

    @functools.partial(
        pl.kernel,
        out_type=(
            jax.ShapeDtypeStruct((B,), jnp.float32),
            jax.ShapeDtypeStruct((NW * NCHUNK, NEG, CH), jnp.float32),
        ),
        mesh=mesh,
        compiler_params=pltpu.CompilerParams(needs_layout_passes=False),
        scratch_types=[
            pltpu.VMEM((EPW,), jnp.int32),
            pltpu.VMEM((EPW,), jnp.int32),
            pltpu.VMEM((EPW * NEG,), jnp.int32),
            pltpu.VMEM((2, CH, D), jnp.float32),
            pltpu.VMEM((2, CH, D), jnp.float32),
            pltpu.VMEM((2, CH * NEG, D), jnp.float32),
            pltpu.VMEM((CH,), jnp.float32),
            pltpu.VMEM((NEG, CH), jnp.float32),
            pltpu.SemaphoreType.DMA,
            pltpu.SemaphoreType.DMA,
        ],
    )
    def kern(pos_u_h, pos_v_h, neg_h, u_h, v_h, pos_o, neg_o,
             idx_u, idx_v, idx_n, rows_u, rows_v, rows_n, out_p, out_n,
             sem0, sem1):
        wid = lax.axis_index("s") * NC + lax.axis_index("c")
        wbase = pl.multiple_of(wid * EPW, EPW)
        pltpu.sync_copy(pos_u_h.at[pl.ds(wbase, EPW)], idx_u)
        pltpu.sync_copy(pos_v_h.at[pl.ds(wbase, EPW)], idx_v)
        pltpu.sync_copy(neg_h.at[pl.ds(wbase * NEG, EPW * NEG)], idx_n)
        sems = (sem0, sem1)

        def fire(c, b):
            off = pl.multiple_of(c * CH, CH)
            sem = sems[b]
            iu = idx_u[pl.ds(off, 16)]
            iv = idx_v[pl.ds(off, 16)]
            for r in range(CH):
                pltpu.async_copy(u_h.at[pl.ds(iu[r], 1)],
                                 rows_u.at[b].at[pl.ds(r, 1)], sem)
                pltpu.async_copy(v_h.at[pl.ds(iv[r], 1)],
                                 rows_v.at[b].at[pl.ds(r, 1)], sem)

            @pl.loop(0, NEG)
            def _(q):
                nbase = pl.multiple_of(off * NEG, 16) + q * 16
                iv16 = idx_n[pl.ds(nbase, 16)]
                for r in range(16):
                    pltpu.async_copy(v_h.at[pl.ds(iv16[r], 1)],
                                     rows_n.at[b].at[pl.ds(q * 16 + r, 1)], sem)

        def drain(b):
            sem = sems[b]
            pltpu.make_async_copy(u_h.at[pl.ds(0, CH)], rows_u.at[b], sem).wait()
            pltpu.make_async_copy(u_h.at[pl.ds(0, CH)], rows_v.at[b], sem).wait()
            pltpu.make_async_copy(u_h.at[pl.ds(0, CH * NEG)], rows_n.at[b],
                                  sem).wait()

        def compute(c, b):
            ru, rv, rn = rows_u.at[b], rows_v.at[b], rows_n.at[b]
            ilane = lax.iota(jnp.int32, 16)
            NK = D // 16
            for g in range(NGRP):

                def ebody(e, carry, ru=ru, rv=rv, rn=rn, g=g):
                    accp, accn = carry
                    accn = list(accn)
                    eg = g * 16 + e
                    sel = ilane == e
                    us = [ru[eg, pl.ds(16 * k, 16)] for k in range(NK)]
                    vs = [rv[eg, pl.ds(16 * k, 16)] for k in range(NK)]
                    pp = us[0] * vs[0]
                    for k in range(1, NK):
                        pp = pp + us[k] * vs[k]
                    accp = jnp.where(sel, jnp.sum(pp), accp)
                    for n in range(NEG):
                        ns = [rn[eg * NEG + n, pl.ds(16 * k, 16)]
                              for k in range(NK)]
                        qq = us[0] * ns[0]
                        for k in range(1, NK):
                            qq = qq + us[k] * ns[k]
                        accn[n] = jnp.where(sel, jnp.sum(qq), accn[n])
                    return accp, tuple(accn)

                accp, accn = lax.fori_loop(
                    0, 16, ebody,
                    (jnp.zeros((16,), jnp.float32),
                     tuple(jnp.zeros((16,), jnp.float32) for _ in range(NEG))))
                out_p[pl.ds(g * 16, 16)] = accp
                for n in range(NEG):
                    out_n[n, pl.ds(g * 16, 16)] = accn[n]
            pltpu.sync_copy(out_p, pos_o.at[pl.ds(wbase + c * CH, CH)])
            pltpu.sync_copy(out_n, neg_o.at[wid * NCHUNK + c])

        fire(0, 0)

        @pl.loop(0, NCHUNK, step=2)
        def _(c):
            drain(0)
            fire(c + 1, 1)
            compute(c, 0)
            drain(1)

            @pl.when(c + 2 < NCHUNK)
            def _():
                fire(c + 2, 0)

            compute(c + 1, 1)

    return kern(pos_u, pos_v, neg_v_flat, u_emb, v_emb)


def _loss_body(pos_ref, neg_ref, out_ref):
    p = jnp.clip(pos_ref[...], -10.0, 10.0)
    lp = jnp.sum(jnp.log1p(jnp.exp(-p)))
    q = jnp.clip(neg_ref[...], -10.0, 10.0)
    ln = jnp.sum(jnp.log1p(jnp.exp(q)))
    out_ref[0, 0] = (lp + ln) * (1.0 / B)


def kernel(pos_u, pos_v, neg_v, u_emb, v_emb):
    pos_s, neg_s = _sc_scores(pos_u, pos_v, neg_v.reshape(-1), u_emb, v_emb)
    out = pl.pallas_call(
        _loss_body,
        out_shape=jax.ShapeDtypeStruct((1, 1), jnp.float32),
        out_specs=pl.BlockSpec(memory_space=pltpu.SMEM),
    )(pos_s.reshape(128, 128), neg_s.reshape(NW * NCHUNK * NEG, CH))
    return out[0, 0]
